# baseline (speedup 1.0000x reference)
import functools

import jax
import jax.numpy as jnp
from jax import lax
from jax.experimental import pallas as pl
from jax.experimental.pallas import tpu as pltpu
from jax.experimental.pallas import tpu_sc as plsc

N = 10000
E = 320000
D = 128
CELL_LEN = 100
SAME_W = 0.3
CROSS_W = 1.0

NC = 2
NS = 16
NW = NC * NS
EPW = E // NW
C = 80
NCHUNK = EPW // C
N_PAD = 10240
ZPT = N_PAD // NS
LANES = 16
NBUF = 3


@functools.lru_cache(maxsize=None)
def _sc_agg(with_counts):
  mesh = plsc.VectorSubcoreMesh(core_axis_name="c", subcore_axis_name="s")
  out_type = [jax.ShapeDtypeStruct((NC, N_PAD, D), jnp.float32)]
  if with_counts:
    out_type.append(jax.ShapeDtypeStruct((NC, N_PAD), jnp.float32))
  scratch = [
      pltpu.VMEM_SHARED((N_PAD, D), jnp.float32),
      pltpu.VMEM_SHARED((N_PAD,), jnp.float32),
      pltpu.VMEM((NBUF * C, D), jnp.float32),
      pltpu.VMEM((3 * NBUF, C), jnp.int32),
      pltpu.VMEM((NBUF, C), jnp.int32),
      pltpu.VMEM((ZPT,), jnp.float32),
  ] + [pltpu.SemaphoreType.DMA for _ in range(4 * NBUF)]

  def body(epk_h, x_h, *rest):
    if with_counts:
      out_acc, out_cnt = rest[0], rest[1]
      scr = rest[2:]
    else:
      out_acc = rest[0]
      scr = rest[1:]
    acc_s, cnt_s, rows_p, ebuf, cidx, zo_v = scr[:6]
    sems = scr[6:]
    se = sems[:NBUF]
    sg = sems[NBUF:2 * NBUF]
    ss = sems[2 * NBUF:3 * NBUF]
    sc = sems[3 * NBUF:4 * NBUF]

    cid = lax.axis_index("c")
    sid = lax.axis_index("s")
    wid = sid * NC + cid

    def zrow(i, carry):
      for d in range(D // LANES):
        rows_p[i, pl.ds(d * LANES, LANES)] = jnp.zeros((LANES,), jnp.float32)
      return carry

    lax.fori_loop(0, NBUF * C, zrow, 0)
    for j in range(ZPT // LANES):
      zo_v[pl.ds(j * LANES, LANES)] = jnp.zeros((LANES,), jnp.float32)

    lo = sid * ZPT
    nz = NBUF * C
    pltpu.sync_copy(rows_p, acc_s.at[pl.ds(lo, nz), :])
    pltpu.sync_copy(rows_p, acc_s.at[pl.ds(lo + nz, nz), :])
    pltpu.sync_copy(rows_p.at[pl.ds(0, ZPT - 2 * nz), :],
                    acc_s.at[pl.ds(lo + 2 * nz, ZPT - 2 * nz), :])
    if with_counts:
      pltpu.sync_copy(zo_v, cnt_s.at[pl.ds(lo, ZPT)])
      for j in range(C // LANES):
        zo_v[pl.ds(j * LANES, LANES)] = jnp.ones((LANES,), jnp.float32)
    plsc.subcore_barrier()

    def rbuf(b):
      return rows_p.at[pl.ds(b * C, C), :]

    def ones_v():
      return zo_v.at[pl.ds(0, C)]

    def e_start(k, b):
      pltpu.async_copy(epk_h.at[wid, k], ebuf.at[pl.ds(3 * b, 3), :], se[b])

    def e_wait(k, b):
      pltpu.make_async_copy(
          epk_h.at[wid, k], ebuf.at[pl.ds(3 * b, 3), :], se[b]).wait()

    def g_start(k, b):
      pltpu.async_copy(x_h.at[ebuf.at[3 * b]], rbuf(b), sg[b])

    def g_wait(k, b):
      pltpu.make_async_copy(x_h.at[ebuf.at[3 * b]], rbuf(b), sg[b]).wait()

    def s_start(k, b):
      pltpu.async_copy(rbuf(b), acc_s.at[cidx.at[b]], ss[b], add=True)
      if with_counts:
        pltpu.async_copy(ones_v(), cnt_s.at[cidx.at[b]], sc[b], add=True)

    def s_wait(k, b):
      pltpu.make_async_copy(rbuf(b), acc_s.at[cidx.at[b]], ss[b]).wait()
      if with_counts:
        pltpu.make_async_copy(ones_v(), cnt_s.at[cidx.at[b]], sc[b]).wait()

    def scale(k, b):
      def jbody(j, carry):
        sl = pl.ds(j * LANES, LANES)
        r16 = ebuf[3 * b, sl]
        c16 = ebuf[3 * b + 1, sl]
        ew16 = lax.bitcast_convert_type(ebuf[3 * b + 2, sl], jnp.float32)
        cidx[b, sl] = c16
        rt = jnp.where(r16 <= CELL_LEN, jnp.float32(1.0), jnp.float32(0.0))
        ct = jnp.where(c16 <= CELL_LEN, jnp.float32(1.0), jnp.float32(0.0))
        diff = jnp.abs(rt - ct)
        w16 = (jnp.float32(SAME_W)
               + jnp.float32(CROSS_W - SAME_W) * diff) * ew16
        for l in range(LANES):
          wr = w16[l]
          r = b * C + j * LANES + l
          for d in range(D // LANES):
            s2 = pl.ds(d * LANES, LANES)
            rows_p[r, s2] = rows_p[r, s2] * wr
        return carry

      lax.fori_loop(0, C // LANES, jbody, 0)

    def step(k, b, wait_prev, load_next2):
      if wait_prev:
        s_wait(k - 2, (b + 1) % NBUF)
      if load_next2:
        e_start(k + 2, (b + 2) % NBUF)
      e_wait(k + 1, (b + 1) % NBUF)
      g_start(k + 1, (b + 1) % NBUF)
      g_wait(k, b)
      scale(k, b)
      s_start(k, b)

    e_start(0, 0)
    e_start(1, 1)
    e_wait(0, 0)
    g_start(0, 0)
    step(0, 0, False, True)
    step(1, 1, False, True)
    step(2, 2, True, True)

    def tri_body(p, carry):
      k = 3 * p
      step(k, 0, True, True)
      step(k + 1, 1, True, True)
      step(k + 2, 2, True, True)
      return carry

    lax.fori_loop(1, (NCHUNK - 2) // 3, tri_body, 0)

    s_wait(121, 1)
    e_wait(124, 1)
    g_start(124, 1)
    g_wait(123, 0)
    scale(123, 0)
    s_start(123, 0)
    s_wait(122, 2)
    g_wait(124, 1)
    scale(124, 1)
    s_start(124, 1)
    s_wait(123, 0)
    s_wait(124, 1)

    plsc.subcore_barrier()

    pltpu.sync_copy(acc_s.at[pl.ds(lo, ZPT), :], out_acc.at[cid, pl.ds(lo, ZPT), :])
    if with_counts:
      pltpu.sync_copy(cnt_s.at[pl.ds(lo, ZPT)], out_cnt.at[cid, pl.ds(lo, ZPT)])

  return pl.kernel(body, out_type=out_type, mesh=mesh, scratch_types=scratch)


BN = 2000


def _mm_body(x_ref, w_ref, o_ref):
  o_ref[...] = lax.dot_general(
      x_ref[...], w_ref[...], (((1,), (1,)), ((), ())),
      preferred_element_type=jnp.float32)


def _tc_mm(x, w):
  return pl.pallas_call(
      _mm_body,
      grid=(N // BN,),
      in_specs=[
          pl.BlockSpec((BN, D), lambda i: (i, 0)),
          pl.BlockSpec((D, D), lambda i: (0, 0)),
      ],
      out_specs=pl.BlockSpec((BN, D), lambda i: (i, 0)),
      out_shape=jax.ShapeDtypeStruct((N, D), jnp.float32),
  )(x, w)


def _add_body(acc_ref, cnt_ref, m_ref, o_ref):
  acc = acc_ref[0] + acc_ref[1]
  cnt = cnt_ref[0] + cnt_ref[1]
  inv = 1.0 / jnp.maximum(cnt, 1.0)
  o_ref[...] = acc * inv + m_ref[...]


def _tc_add(acc_p, cnt_p, m):
  return pl.pallas_call(
      _add_body,
      grid=(N // BN,),
      in_specs=[
          pl.BlockSpec((NC, BN, D), lambda i: (0, i, 0)),
          pl.BlockSpec((NC, BN, 1), lambda i: (0, i, 0)),
          pl.BlockSpec((BN, D), lambda i: (i, 0)),
      ],
      out_specs=pl.BlockSpec((BN, D), lambda i: (i, 0)),
      out_shape=jax.ShapeDtypeStruct((N, D), jnp.float32),
  )(acc_p, cnt_p, m)


def kernel(x, edge_index, edge_weight, W1, W2):
  eir = edge_index.reshape(2, NW, NCHUNK, C)
  ewr = lax.bitcast_convert_type(edge_weight, jnp.int32).reshape(NW, NCHUNK, C)
  epk = jnp.stack([eir[0], eir[1], ewr], axis=2)

  m1 = _tc_mm(x, W1)
  acc_p, cnt_p = _sc_agg(True)(epk, x)
  cnt_p3 = cnt_p.reshape(NC, N_PAD, 1)
  h1 = _tc_add(acc_p, cnt_p3, m1)

  m2 = _tc_mm(h1, W2)
  (acc_p2,) = _sc_agg(False)(epk, h1)
  h2 = _tc_add(acc_p2, cnt_p3, m2)
  return h2


# speedup vs baseline: 1.0956x; 1.0956x over previous
import functools

import jax
import jax.numpy as jnp
from jax import lax
from jax.experimental import pallas as pl
from jax.experimental.pallas import tpu as pltpu
from jax.experimental.pallas import tpu_sc as plsc

N = 10000
E = 320000
D = 128
CELL_LEN = 100
SAME_W = 0.3
CROSS_W = 1.0

NC = 2
NS = 16
NW = NC * NS
EPW = E // NW
C = 80
NCHUNK = EPW // C
N_PAD = 10240
ZPT = N_PAD // NS
LANES = 16
NBUF = 3


@functools.lru_cache(maxsize=None)
def _sc_agg(with_counts):
  mesh = plsc.VectorSubcoreMesh(core_axis_name="c", subcore_axis_name="s")
  out_type = [jax.ShapeDtypeStruct((NC, N_PAD, D), jnp.float32)]
  if with_counts:
    out_type.append(jax.ShapeDtypeStruct((NC, N_PAD), jnp.float32))
  scratch = [
      pltpu.VMEM_SHARED((N_PAD, D), jnp.float32),
      pltpu.VMEM_SHARED((N_PAD,), jnp.float32),
      pltpu.VMEM((NBUF * C, D), jnp.float32),
      pltpu.VMEM((NBUF, C), jnp.int32),
      pltpu.VMEM((NBUF, C), jnp.int32),
      pltpu.VMEM((NBUF, C), jnp.float32),
      pltpu.VMEM((NBUF, C), jnp.int32),
      pltpu.VMEM((ZPT,), jnp.float32),
  ] + [pltpu.SemaphoreType.DMA for _ in range(4 * NBUF)]

  def body(row_h, col_h, ew_h, x_h, *rest):
    if with_counts:
      out_acc, out_cnt = rest[0], rest[1]
      scr = rest[2:]
    else:
      out_acc = rest[0]
      scr = rest[1:]
    acc_s, cnt_s, rows_p, rib, cib, ewb, cidx, zo_v = scr[:8]
    sems = scr[8:]
    se = sems[:NBUF]
    sg = sems[NBUF:2 * NBUF]
    ss = sems[2 * NBUF:3 * NBUF]
    sc = sems[3 * NBUF:4 * NBUF]

    cid = lax.axis_index("c")
    sid = lax.axis_index("s")
    wid = sid * NC + cid
    ebase = wid * EPW

    def zrow(i, carry):
      for d in range(D // LANES):
        rows_p[i, pl.ds(d * LANES, LANES)] = jnp.zeros((LANES,), jnp.float32)
      return carry

    lax.fori_loop(0, NBUF * C, zrow, 0)
    for j in range(ZPT // LANES):
      zo_v[pl.ds(j * LANES, LANES)] = jnp.zeros((LANES,), jnp.float32)

    lo = sid * ZPT
    nz = NBUF * C
    pltpu.sync_copy(rows_p, acc_s.at[pl.ds(lo, nz), :])
    pltpu.sync_copy(rows_p, acc_s.at[pl.ds(lo + nz, nz), :])
    pltpu.sync_copy(rows_p.at[pl.ds(0, ZPT - 2 * nz), :],
                    acc_s.at[pl.ds(lo + 2 * nz, ZPT - 2 * nz), :])
    if with_counts:
      pltpu.sync_copy(zo_v, cnt_s.at[pl.ds(lo, ZPT)])
      for j in range(C // LANES):
        zo_v[pl.ds(j * LANES, LANES)] = jnp.ones((LANES,), jnp.float32)
    plsc.subcore_barrier()

    def rbuf(b):
      return rows_p.at[pl.ds(b * C, C), :]

    def ones_v():
      return zo_v.at[pl.ds(0, C)]

    def e_start(k, b):
      base = pl.multiple_of(ebase + k * C, 8)
      pltpu.async_copy(row_h.at[pl.ds(base, C)], rib.at[b], se[b])
      pltpu.async_copy(col_h.at[pl.ds(base, C)], cib.at[b], se[b])
      pltpu.async_copy(ew_h.at[pl.ds(base, C)], ewb.at[b], se[b])

    def e_wait(k, b):
      base = pl.multiple_of(ebase + k * C, 8)
      pltpu.make_async_copy(row_h.at[pl.ds(base, C)], rib.at[b], se[b]).wait()
      pltpu.make_async_copy(col_h.at[pl.ds(base, C)], cib.at[b], se[b]).wait()
      pltpu.make_async_copy(ew_h.at[pl.ds(base, C)], ewb.at[b], se[b]).wait()

    def g_start(k, b):
      pltpu.async_copy(x_h.at[rib.at[b]], rbuf(b), sg[b])

    def g_wait(k, b):
      pltpu.make_async_copy(x_h.at[rib.at[b]], rbuf(b), sg[b]).wait()

    def s_start(k, b):
      pltpu.async_copy(rbuf(b), acc_s.at[cidx.at[b]], ss[b], add=True)
      if with_counts:
        pltpu.async_copy(ones_v(), cnt_s.at[cidx.at[b]], sc[b], add=True)

    def s_wait(k, b):
      pltpu.make_async_copy(rbuf(b), acc_s.at[cidx.at[b]], ss[b]).wait()
      if with_counts:
        pltpu.make_async_copy(ones_v(), cnt_s.at[cidx.at[b]], sc[b]).wait()

    def scale(k, b):
      def jbody(j, carry):
        sl = pl.ds(j * LANES, LANES)
        r16 = rib[b, sl]
        c16 = cib[b, sl]
        ew16 = ewb[b, sl]
        cidx[b, sl] = c16
        rt = jnp.where(r16 <= CELL_LEN, jnp.float32(1.0), jnp.float32(0.0))
        ct = jnp.where(c16 <= CELL_LEN, jnp.float32(1.0), jnp.float32(0.0))
        diff = jnp.abs(rt - ct)
        w16 = (jnp.float32(SAME_W)
               + jnp.float32(CROSS_W - SAME_W) * diff) * ew16
        for l in range(LANES):
          wr = w16[l]
          r = b * C + j * LANES + l
          for d in range(D // LANES):
            s2 = pl.ds(d * LANES, LANES)
            rows_p[r, s2] = rows_p[r, s2] * wr
        return carry

      lax.fori_loop(0, C // LANES, jbody, 0)

    def step(k, b, wait_prev, load_next2):
      if wait_prev:
        s_wait(k - 2, (b + 1) % NBUF)
      if load_next2:
        e_start(k + 2, (b + 2) % NBUF)
      e_wait(k + 1, (b + 1) % NBUF)
      g_start(k + 1, (b + 1) % NBUF)
      g_wait(k, b)
      scale(k, b)
      s_start(k, b)

    e_start(0, 0)
    e_start(1, 1)
    e_wait(0, 0)
    g_start(0, 0)
    step(0, 0, False, True)
    step(1, 1, False, True)
    step(2, 2, True, True)

    def tri_body(p, carry):
      k = 3 * p
      step(k, 0, True, True)
      step(k + 1, 1, True, True)
      step(k + 2, 2, True, True)
      return carry

    lax.fori_loop(1, (NCHUNK - 2) // 3, tri_body, 0)

    s_wait(121, 1)
    e_wait(124, 1)
    g_start(124, 1)
    g_wait(123, 0)
    scale(123, 0)
    s_start(123, 0)
    s_wait(122, 2)
    g_wait(124, 1)
    scale(124, 1)
    s_start(124, 1)
    s_wait(123, 0)
    s_wait(124, 1)

    plsc.subcore_barrier()

    pltpu.sync_copy(acc_s.at[pl.ds(lo, ZPT), :], out_acc.at[cid, pl.ds(lo, ZPT), :])
    if with_counts:
      pltpu.sync_copy(cnt_s.at[pl.ds(lo, ZPT)], out_cnt.at[cid, pl.ds(lo, ZPT)])

  return pl.kernel(body, out_type=out_type, mesh=mesh, scratch_types=scratch)


BN = 2000


def _addmm_body(acc_ref, cnt_ref, x_ref, w1_ref, w2_ref, h_ref, m2_ref):
  acc = acc_ref[0] + acc_ref[1]
  cnt = cnt_ref[0] + cnt_ref[1]
  inv = 1.0 / jnp.maximum(cnt, 1.0)
  h = acc * inv + lax.dot_general(
      x_ref[...], w1_ref[...], (((1,), (1,)), ((), ())),
      preferred_element_type=jnp.float32)
  h_ref[...] = h
  m2_ref[...] = lax.dot_general(
      h, w2_ref[...], (((1,), (1,)), ((), ())),
      preferred_element_type=jnp.float32)


def _tc_addmm(acc_p, cnt_p, x, w1, w2):
  return pl.pallas_call(
      _addmm_body,
      grid=(N // BN,),
      in_specs=[
          pl.BlockSpec((NC, BN, D), lambda i: (0, i, 0)),
          pl.BlockSpec((NC, BN, 1), lambda i: (0, i, 0)),
          pl.BlockSpec((BN, D), lambda i: (i, 0)),
          pl.BlockSpec((D, D), lambda i: (0, 0)),
          pl.BlockSpec((D, D), lambda i: (0, 0)),
      ],
      out_specs=[
          pl.BlockSpec((BN, D), lambda i: (i, 0)),
          pl.BlockSpec((BN, D), lambda i: (i, 0)),
      ],
      out_shape=[
          jax.ShapeDtypeStruct((N, D), jnp.float32),
          jax.ShapeDtypeStruct((N, D), jnp.float32),
      ],
  )(acc_p, cnt_p, x, w1, w2)


def _add_body(acc_ref, cnt_ref, m_ref, o_ref):
  acc = acc_ref[0] + acc_ref[1]
  cnt = cnt_ref[0] + cnt_ref[1]
  inv = 1.0 / jnp.maximum(cnt, 1.0)
  o_ref[...] = acc * inv + m_ref[...]


def _tc_add(acc_p, cnt_p, m):
  return pl.pallas_call(
      _add_body,
      grid=(N // BN,),
      in_specs=[
          pl.BlockSpec((NC, BN, D), lambda i: (0, i, 0)),
          pl.BlockSpec((NC, BN, 1), lambda i: (0, i, 0)),
          pl.BlockSpec((BN, D), lambda i: (i, 0)),
      ],
      out_specs=pl.BlockSpec((BN, D), lambda i: (i, 0)),
      out_shape=jax.ShapeDtypeStruct((N, D), jnp.float32),
  )(acc_p, cnt_p, m)


def kernel(x, edge_index, edge_weight, W1, W2):
  row = edge_index[0]
  col = edge_index[1]
  acc_p, cnt_p = _sc_agg(True)(row, col, edge_weight, x)
  cnt_p3 = cnt_p.reshape(NC, N_PAD, 1)
  h1, m2 = _tc_addmm(acc_p, cnt_p3, x, W1, W2)

  (acc_p2,) = _sc_agg(False)(row, col, edge_weight, h1)
  h2 = _tc_add(acc_p2, cnt_p3, m2)
  return h2


# speedup vs baseline: 1.0991x; 1.0032x over previous
import functools

import jax
import jax.numpy as jnp
from jax import lax
from jax.experimental import pallas as pl
from jax.experimental.pallas import tpu as pltpu
from jax.experimental.pallas import tpu_sc as plsc

N = 10000
E = 320000
D = 128
CELL_LEN = 100
SAME_W = 0.3
CROSS_W = 1.0

NC = 2
NS = 16
NW = NC * NS
EPW = E // NW
C = 80
NCHUNK = EPW // C
N_PAD = 10240
ZPT = N_PAD // NS
LANES = 16
NBUF = 3


@functools.lru_cache(maxsize=None)
def _sc_agg(with_counts):
  mesh = plsc.VectorSubcoreMesh(core_axis_name="c", subcore_axis_name="s")
  out_type = [jax.ShapeDtypeStruct((NC, N_PAD, D), jnp.float32)]
  if with_counts:
    out_type.append(jax.ShapeDtypeStruct((NC, N_PAD), jnp.float32))
  scratch = [
      pltpu.VMEM_SHARED((N_PAD, D), jnp.float32),
      pltpu.VMEM_SHARED((N_PAD,), jnp.float32),
      pltpu.VMEM((NBUF * C, D), jnp.float32),
      pltpu.VMEM((NBUF, C), jnp.int32),
      pltpu.VMEM((NBUF, C), jnp.int32),
      pltpu.VMEM((NBUF, C), jnp.float32),
      pltpu.VMEM((NBUF, C), jnp.int32),
      pltpu.VMEM((ZPT,), jnp.float32),
  ] + [pltpu.SemaphoreType.DMA for _ in range(4 * NBUF)]

  def body(row_h, col_h, ew_h, x_h, *rest):
    if with_counts:
      out_acc, out_cnt = rest[0], rest[1]
      scr = rest[2:]
    else:
      out_acc = rest[0]
      scr = rest[1:]
    acc_s, cnt_s, rows_p, rib, cib, ewb, cidx, zo_v = scr[:8]
    sems = scr[8:]
    se = sems[:NBUF]
    sg = sems[NBUF:2 * NBUF]
    ss = sems[2 * NBUF:3 * NBUF]
    sc = sems[3 * NBUF:4 * NBUF]

    cid = lax.axis_index("c")
    sid = lax.axis_index("s")
    wid = sid * NC + cid
    ebase = wid * EPW

    def zrow(i, carry):
      for d in range(D // LANES):
        rows_p[i, pl.ds(d * LANES, LANES)] = jnp.zeros((LANES,), jnp.float32)
      return carry

    lax.fori_loop(0, NBUF * C, zrow, 0)
    for j in range(ZPT // LANES):
      zo_v[pl.ds(j * LANES, LANES)] = jnp.zeros((LANES,), jnp.float32)

    lo = sid * ZPT
    nz = NBUF * C
    pltpu.sync_copy(rows_p, acc_s.at[pl.ds(lo, nz), :])
    pltpu.sync_copy(rows_p, acc_s.at[pl.ds(lo + nz, nz), :])
    pltpu.sync_copy(rows_p.at[pl.ds(0, ZPT - 2 * nz), :],
                    acc_s.at[pl.ds(lo + 2 * nz, ZPT - 2 * nz), :])
    if with_counts:
      pltpu.sync_copy(zo_v, cnt_s.at[pl.ds(lo, ZPT)])
      for j in range(C // LANES):
        zo_v[pl.ds(j * LANES, LANES)] = jnp.ones((LANES,), jnp.float32)
    plsc.subcore_barrier()

    def rbuf(b):
      return rows_p.at[pl.ds(b * C, C), :]

    def ones_v():
      return zo_v.at[pl.ds(0, C)]

    def e_start(k, b):
      base = pl.multiple_of(ebase + k * C, 8)
      pltpu.async_copy(row_h.at[pl.ds(base, C)], rib.at[b], se[b])
      pltpu.async_copy(col_h.at[pl.ds(base, C)], cib.at[b], se[b])
      pltpu.async_copy(ew_h.at[pl.ds(base, C)], ewb.at[b], se[b])

    def e_wait(k, b):
      base = pl.multiple_of(ebase + k * C, 8)
      pltpu.make_async_copy(row_h.at[pl.ds(base, C)], rib.at[b], se[b]).wait()
      pltpu.make_async_copy(col_h.at[pl.ds(base, C)], cib.at[b], se[b]).wait()
      pltpu.make_async_copy(ew_h.at[pl.ds(base, C)], ewb.at[b], se[b]).wait()

    def g_start(k, b):
      pltpu.async_copy(x_h.at[rib.at[b]], rbuf(b), sg[b])

    def g_wait(k, b):
      pltpu.make_async_copy(x_h.at[rib.at[b]], rbuf(b), sg[b]).wait()

    def s_start(k, b):
      pltpu.async_copy(rbuf(b), acc_s.at[cidx.at[b]], ss[b], add=True)
      if with_counts:
        pltpu.async_copy(ones_v(), cnt_s.at[cidx.at[b]], sc[b], add=True)

    def s_wait(k, b):
      pltpu.make_async_copy(rbuf(b), acc_s.at[cidx.at[b]], ss[b]).wait()
      if with_counts:
        pltpu.make_async_copy(ones_v(), cnt_s.at[cidx.at[b]], sc[b]).wait()

    def scale(k, b):
      def jbody(j, carry):
        sl = pl.ds(j * LANES, LANES)
        r16 = rib[b, sl]
        c16 = cib[b, sl]
        ew16 = ewb[b, sl]
        cidx[b, sl] = c16
        rt = jnp.where(r16 <= CELL_LEN, jnp.float32(1.0), jnp.float32(0.0))
        ct = jnp.where(c16 <= CELL_LEN, jnp.float32(1.0), jnp.float32(0.0))
        diff = jnp.abs(rt - ct)
        w16 = (jnp.float32(SAME_W)
               + jnp.float32(CROSS_W - SAME_W) * diff) * ew16
        for l in range(LANES):
          wr = w16[l]
          r = b * C + j * LANES + l
          for d in range(D // LANES):
            s2 = pl.ds(d * LANES, LANES)
            rows_p[r, s2] = rows_p[r, s2] * wr
        return carry

      lax.fori_loop(0, C // LANES, jbody, 0)

    def step(k, b, wait_prev, load_next2):
      if wait_prev:
        s_wait(k - 2, (b + 1) % NBUF)
      if load_next2:
        e_start(k + 2, (b + 2) % NBUF)
      e_wait(k + 1, (b + 1) % NBUF)
      g_start(k + 1, (b + 1) % NBUF)
      g_wait(k, b)
      scale(k, b)
      s_start(k, b)

    e_start(0, 0)
    e_start(1, 1)
    e_wait(0, 0)
    g_start(0, 0)
    step(0, 0, False, True)
    step(1, 1, False, True)
    step(2, 2, True, True)

    def tri_body(p, carry):
      k = 3 * p
      step(k, 0, True, True)
      step(k + 1, 1, True, True)
      step(k + 2, 2, True, True)
      return carry

    lax.fori_loop(1, (NCHUNK - 2) // 3, tri_body, 0)

    s_wait(121, 1)
    e_wait(124, 1)
    g_start(124, 1)
    g_wait(123, 0)
    scale(123, 0)
    s_start(123, 0)
    s_wait(122, 2)
    g_wait(124, 1)
    scale(124, 1)
    s_start(124, 1)
    s_wait(123, 0)
    s_wait(124, 1)

    plsc.subcore_barrier()

    pltpu.sync_copy(acc_s.at[pl.ds(lo, ZPT), :], out_acc.at[cid, pl.ds(lo, ZPT), :])
    if with_counts:
      pltpu.sync_copy(cnt_s.at[pl.ds(lo, ZPT)], out_cnt.at[cid, pl.ds(lo, ZPT)])

  return pl.kernel(body, out_type=out_type, mesh=mesh, scratch_types=scratch)


BN = 2000


def _mm_body(x_ref, w_ref, o_ref):
  o_ref[...] = lax.dot_general(
      x_ref[...], w_ref[...], (((1,), (1,)), ((), ())),
      preferred_element_type=jnp.float32)


def _tc_mm(x, w):
  return pl.pallas_call(
      _mm_body,
      grid=(N // BN,),
      in_specs=[
          pl.BlockSpec((BN, D), lambda i: (i, 0)),
          pl.BlockSpec((D, D), lambda i: (0, 0)),
      ],
      out_specs=pl.BlockSpec((BN, D), lambda i: (i, 0)),
      out_shape=jax.ShapeDtypeStruct((N, D), jnp.float32),
  )(x, w)


def _addmm_body(acc_ref, cnt_ref, m1_ref, w2_ref, h_ref, m2_ref):
  acc = acc_ref[0] + acc_ref[1]
  cnt = cnt_ref[0] + cnt_ref[1]
  inv = 1.0 / jnp.maximum(cnt, 1.0)
  h = acc * inv + m1_ref[...]
  h_ref[...] = h
  m2_ref[...] = lax.dot_general(
      h, w2_ref[...], (((1,), (1,)), ((), ())),
      preferred_element_type=jnp.float32)


def _tc_addmm(acc_p, cnt_p, m1, w2):
  return pl.pallas_call(
      _addmm_body,
      grid=(N // BN,),
      in_specs=[
          pl.BlockSpec((NC, BN, D), lambda i: (0, i, 0)),
          pl.BlockSpec((NC, BN, 1), lambda i: (0, i, 0)),
          pl.BlockSpec((BN, D), lambda i: (i, 0)),
          pl.BlockSpec((D, D), lambda i: (0, 0)),
      ],
      out_specs=[
          pl.BlockSpec((BN, D), lambda i: (i, 0)),
          pl.BlockSpec((BN, D), lambda i: (i, 0)),
      ],
      out_shape=[
          jax.ShapeDtypeStruct((N, D), jnp.float32),
          jax.ShapeDtypeStruct((N, D), jnp.float32),
      ],
  )(acc_p, cnt_p, m1, w2)


def _add_body(acc_ref, cnt_ref, m_ref, o_ref):
  acc = acc_ref[0] + acc_ref[1]
  cnt = cnt_ref[0] + cnt_ref[1]
  inv = 1.0 / jnp.maximum(cnt, 1.0)
  o_ref[...] = acc * inv + m_ref[...]


def _tc_add(acc_p, cnt_p, m):
  return pl.pallas_call(
      _add_body,
      grid=(N // BN,),
      in_specs=[
          pl.BlockSpec((NC, BN, D), lambda i: (0, i, 0)),
          pl.BlockSpec((NC, BN, 1), lambda i: (0, i, 0)),
          pl.BlockSpec((BN, D), lambda i: (i, 0)),
      ],
      out_specs=pl.BlockSpec((BN, D), lambda i: (i, 0)),
      out_shape=jax.ShapeDtypeStruct((N, D), jnp.float32),
  )(acc_p, cnt_p, m)


def kernel(x, edge_index, edge_weight, W1, W2):
  row = edge_index[0]
  col = edge_index[1]
  m1 = _tc_mm(x, W1)
  acc_p, cnt_p = _sc_agg(True)(row, col, edge_weight, x)
  cnt_p3 = cnt_p.reshape(NC, N_PAD, 1)
  h1, m2 = _tc_addmm(acc_p, cnt_p3, m1, W2)

  (acc_p2,) = _sc_agg(False)(row, col, edge_weight, h1)
  h2 = _tc_add(acc_p2, cnt_p3, m2)
  return h2


# speedup vs baseline: 1.1392x; 1.0365x over previous
import functools

import jax
import jax.numpy as jnp
from jax import lax
from jax.experimental import pallas as pl
from jax.experimental.pallas import tpu as pltpu
from jax.experimental.pallas import tpu_sc as plsc

N = 10000
E = 320000
D = 128
CELL_LEN = 100
SAME_W = 0.3
CROSS_W = 1.0

NC = 2
NS = 16
NW = NC * NS
EPW = E // NW
C = 80
NCHUNK = EPW // C
N_PAD = 10240
ZPT = N_PAD // NS
LANES = 16
NBUF = 4


@functools.lru_cache(maxsize=None)
def _sc_agg(with_counts):
  mesh = plsc.VectorSubcoreMesh(core_axis_name="c", subcore_axis_name="s")
  out_type = [jax.ShapeDtypeStruct((NC, N_PAD, D), jnp.float32)]
  if with_counts:
    out_type.append(jax.ShapeDtypeStruct((NC, N_PAD), jnp.float32))
  scratch = [
      pltpu.VMEM_SHARED((N_PAD, D), jnp.float32),
      pltpu.VMEM_SHARED((N_PAD,), jnp.float32),
      pltpu.VMEM((NBUF * C, D), jnp.float32),
      pltpu.VMEM((NBUF, C), jnp.int32),
      pltpu.VMEM((NBUF, C), jnp.int32),
      pltpu.VMEM((NBUF, C), jnp.float32),
      pltpu.VMEM((NBUF, C), jnp.int32),
      pltpu.VMEM((ZPT,), jnp.float32),
  ] + [pltpu.SemaphoreType.DMA for _ in range(4 * NBUF)]

  def body(row_h, col_h, ew_h, x_h, *rest):
    if with_counts:
      out_acc, out_cnt = rest[0], rest[1]
      scr = rest[2:]
    else:
      out_acc = rest[0]
      scr = rest[1:]
    acc_s, cnt_s, rows_p, rib, cib, ewb, cidx, zo_v = scr[:8]
    sems = scr[8:]
    se = sems[:NBUF]
    sg = sems[NBUF:2 * NBUF]
    ss = sems[2 * NBUF:3 * NBUF]
    sc = sems[3 * NBUF:4 * NBUF]

    cid = lax.axis_index("c")
    sid = lax.axis_index("s")
    wid = sid * NC + cid
    ebase = wid * EPW

    def zrow(i, carry):
      for d in range(D // LANES):
        rows_p[i, pl.ds(d * LANES, LANES)] = jnp.zeros((LANES,), jnp.float32)
      return carry

    lax.fori_loop(0, NBUF * C, zrow, 0)
    for j in range(ZPT // LANES):
      zo_v[pl.ds(j * LANES, LANES)] = jnp.zeros((LANES,), jnp.float32)

    lo = sid * ZPT
    nz = NBUF * C
    pltpu.sync_copy(rows_p, acc_s.at[pl.ds(lo, nz), :])
    pltpu.sync_copy(rows_p, acc_s.at[pl.ds(lo + nz, nz), :])
    if ZPT > 2 * nz:
      pltpu.sync_copy(rows_p.at[pl.ds(0, ZPT - 2 * nz), :],
                      acc_s.at[pl.ds(lo + 2 * nz, ZPT - 2 * nz), :])
    if with_counts:
      pltpu.sync_copy(zo_v, cnt_s.at[pl.ds(lo, ZPT)])
      for j in range(C // LANES):
        zo_v[pl.ds(j * LANES, LANES)] = jnp.ones((LANES,), jnp.float32)
    plsc.subcore_barrier()

    def rbuf(b):
      return rows_p.at[pl.ds(b * C, C), :]

    def ones_v():
      return zo_v.at[pl.ds(0, C)]

    def e_start(k, b):
      base = pl.multiple_of(ebase + k * C, 8)
      pltpu.async_copy(row_h.at[pl.ds(base, C)], rib.at[b], se[b])
      pltpu.async_copy(col_h.at[pl.ds(base, C)], cib.at[b], se[b])
      pltpu.async_copy(ew_h.at[pl.ds(base, C)], ewb.at[b], se[b])

    def e_wait(k, b):
      base = pl.multiple_of(ebase + k * C, 8)
      pltpu.make_async_copy(row_h.at[pl.ds(base, C)], rib.at[b], se[b]).wait()
      pltpu.make_async_copy(col_h.at[pl.ds(base, C)], cib.at[b], se[b]).wait()
      pltpu.make_async_copy(ew_h.at[pl.ds(base, C)], ewb.at[b], se[b]).wait()

    def g_start(k, b):
      pltpu.async_copy(x_h.at[rib.at[b]], rbuf(b), sg[b])

    def g_wait(k, b):
      pltpu.make_async_copy(x_h.at[rib.at[b]], rbuf(b), sg[b]).wait()

    def s_start(k, b):
      pltpu.async_copy(rbuf(b), acc_s.at[cidx.at[b]], ss[b], add=True)
      if with_counts:
        pltpu.async_copy(ones_v(), cnt_s.at[cidx.at[b]], sc[b], add=True)

    def s_wait(k, b):
      pltpu.make_async_copy(rbuf(b), acc_s.at[cidx.at[b]], ss[b]).wait()
      if with_counts:
        pltpu.make_async_copy(ones_v(), cnt_s.at[cidx.at[b]], sc[b]).wait()

    def scale(k, b):
      def jbody(j, carry):
        sl = pl.ds(j * LANES, LANES)
        r16 = rib[b, sl]
        c16 = cib[b, sl]
        ew16 = ewb[b, sl]
        cidx[b, sl] = c16
        rt = jnp.where(r16 <= CELL_LEN, jnp.float32(1.0), jnp.float32(0.0))
        ct = jnp.where(c16 <= CELL_LEN, jnp.float32(1.0), jnp.float32(0.0))
        diff = jnp.abs(rt - ct)
        w16 = (jnp.float32(SAME_W)
               + jnp.float32(CROSS_W - SAME_W) * diff) * ew16
        for l in range(LANES):
          wr = w16[l]
          r = b * C + j * LANES + l
          for d in range(D // LANES):
            s2 = pl.ds(d * LANES, LANES)
            rows_p[r, s2] = rows_p[r, s2] * wr
        return carry

      lax.fori_loop(0, C // LANES, jbody, 0)

    def step(k, b, wait_prev=True, load3=True, g2=True):
      if wait_prev:
        s_wait(k - 2, (b + 2) % NBUF)
      if load3:
        e_start(k + 3, (b + 3) % NBUF)
      if g2:
        e_wait(k + 2, (b + 2) % NBUF)
        g_start(k + 2, (b + 2) % NBUF)
      g_wait(k, b)
      scale(k, b)
      s_start(k, b)

    e_start(0, 0)
    e_start(1, 1)
    e_start(2, 2)
    e_wait(0, 0)
    g_start(0, 0)
    e_wait(1, 1)
    g_start(1, 1)
    step(0, 0, wait_prev=False)
    step(1, 1, wait_prev=False)
    step(2, 2)
    step(3, 3)

    def quad_body(p, carry):
      k = 4 * p
      step(k, 0)
      step(k + 1, 1)
      step(k + 2, 2)
      step(k + 3, 3)
      return carry

    lax.fori_loop(1, 30, quad_body, 0)

    step(120, 0)
    step(121, 1)
    step(122, 2, load3=False)
    step(123, 3, load3=False, g2=False)
    step(124, 0, load3=False, g2=False)
    s_wait(123, 3)
    s_wait(124, 0)

    plsc.subcore_barrier()

    pltpu.sync_copy(acc_s.at[pl.ds(lo, ZPT), :], out_acc.at[cid, pl.ds(lo, ZPT), :])
    if with_counts:
      pltpu.sync_copy(cnt_s.at[pl.ds(lo, ZPT)], out_cnt.at[cid, pl.ds(lo, ZPT)])

  return pl.kernel(body, out_type=out_type, mesh=mesh, scratch_types=scratch)


BN = 2000


def _mm_body(x_ref, w_ref, o_ref):
  o_ref[...] = lax.dot_general(
      x_ref[...], w_ref[...], (((1,), (1,)), ((), ())),
      preferred_element_type=jnp.float32)


def _tc_mm(x, w):
  return pl.pallas_call(
      _mm_body,
      grid=(N // BN,),
      in_specs=[
          pl.BlockSpec((BN, D), lambda i: (i, 0)),
          pl.BlockSpec((D, D), lambda i: (0, 0)),
      ],
      out_specs=pl.BlockSpec((BN, D), lambda i: (i, 0)),
      out_shape=jax.ShapeDtypeStruct((N, D), jnp.float32),
  )(x, w)


def _addmm_body(acc_ref, cnt_ref, m1_ref, w2_ref, h_ref, m2_ref):
  acc = acc_ref[0] + acc_ref[1]
  cnt = cnt_ref[0] + cnt_ref[1]
  inv = 1.0 / jnp.maximum(cnt, 1.0)
  h = acc * inv + m1_ref[...]
  h_ref[...] = h
  m2_ref[...] = lax.dot_general(
      h, w2_ref[...], (((1,), (1,)), ((), ())),
      preferred_element_type=jnp.float32)


def _tc_addmm(acc_p, cnt_p, m1, w2):
  return pl.pallas_call(
      _addmm_body,
      grid=(N // BN,),
      in_specs=[
          pl.BlockSpec((NC, BN, D), lambda i: (0, i, 0)),
          pl.BlockSpec((NC, BN, 1), lambda i: (0, i, 0)),
          pl.BlockSpec((BN, D), lambda i: (i, 0)),
          pl.BlockSpec((D, D), lambda i: (0, 0)),
      ],
      out_specs=[
          pl.BlockSpec((BN, D), lambda i: (i, 0)),
          pl.BlockSpec((BN, D), lambda i: (i, 0)),
      ],
      out_shape=[
          jax.ShapeDtypeStruct((N, D), jnp.float32),
          jax.ShapeDtypeStruct((N, D), jnp.float32),
      ],
  )(acc_p, cnt_p, m1, w2)


def _add_body(acc_ref, cnt_ref, m_ref, o_ref):
  acc = acc_ref[0] + acc_ref[1]
  cnt = cnt_ref[0] + cnt_ref[1]
  inv = 1.0 / jnp.maximum(cnt, 1.0)
  o_ref[...] = acc * inv + m_ref[...]


def _tc_add(acc_p, cnt_p, m):
  return pl.pallas_call(
      _add_body,
      grid=(N // BN,),
      in_specs=[
          pl.BlockSpec((NC, BN, D), lambda i: (0, i, 0)),
          pl.BlockSpec((NC, BN, 1), lambda i: (0, i, 0)),
          pl.BlockSpec((BN, D), lambda i: (i, 0)),
      ],
      out_specs=pl.BlockSpec((BN, D), lambda i: (i, 0)),
      out_shape=jax.ShapeDtypeStruct((N, D), jnp.float32),
  )(acc_p, cnt_p, m)


def kernel(x, edge_index, edge_weight, W1, W2):
  row = edge_index[0]
  col = edge_index[1]
  m1 = _tc_mm(x, W1)
  acc_p, cnt_p = _sc_agg(True)(row, col, edge_weight, x)
  cnt_p3 = cnt_p.reshape(NC, N_PAD, 1)
  h1, m2 = _tc_addmm(acc_p, cnt_p3, m1, W2)

  (acc_p2,) = _sc_agg(False)(row, col, edge_weight, h1)
  h2 = _tc_add(acc_p2, cnt_p3, m2)
  return h2
